# baseline (speedup 1.0000x reference)
import functools

import jax
import jax.numpy as jnp
from jax.experimental import pallas as pl

K = 16
ROWS = 256


def _fused_kernel(xr_ref, xf_ref, w1_ref, w2_ref, b_ref, o_ref, *, n, d, k):
    xr = xr_ref[0]
    xf = xf_ref[0]
    w1 = w1_ref[...]
    w2 = w2_ref[...]
    bias = b_ref[...]

    r = xr.shape[0]

    xx_r = jnp.sum(xr * xr, axis=1, keepdims=True)
    xx_f = jax.lax.dot_general(
        jnp.ones((1, d), jnp.float32), xf * xf, (((1,), (1,)), ((), ())),
        precision=jax.lax.Precision.HIGHEST,
        preferred_element_type=jnp.float32)
    a = jax.lax.dot_general(
        xr, xf, (((1,), (1,)), ((), ())),
        preferred_element_type=jnp.float32)
    scores = -xx_r - (-2.0 * a) - xx_f

    iota = jax.lax.broadcasted_iota(jnp.int32, (r, n), 1)
    xf_b = xf.astype(jnp.bfloat16)
    w1_b = w1.astype(jnp.bfloat16)

    def body(_, carry):
        scores, acc = carry
        idx = jnp.argmax(scores, axis=1, keepdims=True)
        hot = iota == idx
        onehot = hot.astype(jnp.bfloat16)
        scores = jnp.where(hot, -jnp.inf, scores)
        sel = jax.lax.dot_general(
            onehot, xf_b, (((1,), (0,)), ((), ())),
            preferred_element_type=jnp.float32)
        dif = (sel - xr).astype(jnp.bfloat16)
        v = jax.lax.dot_general(
            dif, w1_b, (((1,), (1,)), ((), ())),
            preferred_element_type=jnp.float32)
        return scores, jnp.maximum(acc, v)

    acc0 = jnp.full((r, w1.shape[0]), -jnp.inf, dtype=jnp.float32)
    _, acc = jax.lax.fori_loop(0, k, body, (scores, acc0))

    glob = jax.lax.dot_general(
        xr, w2, (((1,), (1,)), ((), ())),
        preferred_element_type=jnp.float32)
    o_ref[0] = acc + glob + bias


def kernel(x, W, b):
    B, d, n = x.shape
    d_out = W.shape[0]
    xt = jnp.transpose(x, (0, 2, 1))
    w1 = W[:, :d]
    w2 = W[:, d:]
    b2 = jnp.reshape(b, (1, d_out))

    grid = (B, n // ROWS)
    out = pl.pallas_call(
        functools.partial(_fused_kernel, n=n, d=d, k=K),
        grid=grid,
        in_specs=[
            pl.BlockSpec((1, ROWS, d), lambda bi, ti: (bi, ti, 0)),
            pl.BlockSpec((1, n, d), lambda bi, ti: (bi, 0, 0)),
            pl.BlockSpec((d_out, d), lambda bi, ti: (0, 0)),
            pl.BlockSpec((d_out, d), lambda bi, ti: (0, 0)),
            pl.BlockSpec((1, d_out), lambda bi, ti: (0, 0)),
        ],
        out_specs=pl.BlockSpec((1, ROWS, d_out), lambda bi, ti: (bi, ti, 0)),
        out_shape=jax.ShapeDtypeStruct((B, n, d_out), jnp.float32),
    )(xt, xt, w1, w2, b2)
    return jnp.transpose(out, (0, 2, 1))


# speedup vs baseline: 1.2071x; 1.2071x over previous
import functools

import jax
import jax.numpy as jnp
from jax import lax
from jax.experimental import pallas as pl
from jax.experimental.pallas import tpu as pltpu
from jax.experimental.pallas import tpu_sc as plsc

K = 16
ROWS = 256


def _topk_kernel(xr_ref, xf_ref, w1_ref, w2_ref, b_ref,
                 idx_ref, h_ref, g_ref, *, n, d, k):
    bi = pl.program_id(0)
    xr = xr_ref[0]
    xf = xf_ref[0]
    w1 = w1_ref[...]
    w2 = w2_ref[...]
    bias = b_ref[...]

    r = xr.shape[0]

    xx_r = jnp.sum(xr * xr, axis=1, keepdims=True)
    xx_f = jax.lax.dot_general(
        jnp.ones((1, d), jnp.float32), xf * xf, (((1,), (1,)), ((), ())),
        precision=jax.lax.Precision.HIGHEST,
        preferred_element_type=jnp.float32)
    a = jax.lax.dot_general(
        xr, xf, (((1,), (1,)), ((), ())),
        preferred_element_type=jnp.float32)
    scores = -xx_r - (-2.0 * a) - xx_f

    iota = jax.lax.broadcasted_iota(jnp.int32, (r, n), 1)
    kiota = jax.lax.broadcasted_iota(jnp.int32, (r, k), 1)

    def body(j, carry):
        scores, idxs = carry
        m = jnp.max(scores, axis=1, keepdims=True)
        eq = scores == m
        idx = jnp.min(jnp.where(eq, iota, n), axis=1, keepdims=True)
        scores = jnp.where(eq, -jnp.inf, scores)
        idxs = jnp.where(kiota == j, idx, idxs)
        return scores, idxs

    idxs0 = jnp.zeros((r, k), dtype=jnp.int32)
    _, idxs = jax.lax.fori_loop(0, k, body, (scores, idxs0))

    idx_ref[0] = idxs + bi * n
    h_ref[0] = jax.lax.dot_general(
        xr, w1, (((1,), (1,)), ((), ())),
        preferred_element_type=jnp.float32)
    g_ref[0] = jax.lax.dot_general(
        xr, w2 - w1, (((1,), (1,)), ((), ())),
        preferred_element_type=jnp.float32) + bias


def _make_sc_gather_max(total, d_out, k):
    info = plsc.get_sparse_core_info()
    nw = info.num_cores * info.num_subcores
    per_w = total // nw
    chunk = 8
    n_chunks = per_w // chunk
    mesh = plsc.VectorSubcoreMesh(core_axis_name="c", subcore_axis_name="s")

    @functools.partial(
        pl.kernel, mesh=mesh,
        out_type=jax.ShapeDtypeStruct((total, d_out), jnp.float32),
        scratch_types=[
            pltpu.VMEM((chunk * k,), jnp.int32),
            pltpu.VMEM((chunk * k, d_out), jnp.float32),
            pltpu.VMEM((chunk, d_out), jnp.float32),
            pltpu.VMEM((chunk, d_out), jnp.float32),
            pltpu.SemaphoreType.DMA,
        ],
    )
    def sc_gather_max(idx_hbm, h_hbm, g_hbm, out_hbm,
                      idx_v, rows_v, g_v, out_v, sem):
        wid = lax.axis_index("s") * info.num_cores + lax.axis_index("c")
        base = wid * per_w

        def chunk_body(c, _):
            cbase = base + c * chunk
            pltpu.sync_copy(idx_hbm.at[pl.ds(cbase * k, chunk * k)], idx_v)
            pltpu.async_copy(h_hbm.at[idx_v], rows_v, sem).wait()
            pltpu.sync_copy(g_hbm.at[pl.ds(cbase, chunk)], g_v)

            def point_body(p, _):
                for cg in range(d_out // 16):
                    acc = rows_v[p * k, pl.ds(cg * 16, 16)]
                    for j in range(1, k):
                        acc = jnp.maximum(
                            acc, rows_v[p * k + j, pl.ds(cg * 16, 16)])
                    out_v[p, pl.ds(cg * 16, 16)] = (
                        acc + g_v[p, pl.ds(cg * 16, 16)])
                return 0

            lax.fori_loop(0, chunk, point_body, 0)
            pltpu.sync_copy(out_v, out_hbm.at[pl.ds(cbase, chunk)])
            return 0

        lax.fori_loop(0, n_chunks, chunk_body, 0)

    return sc_gather_max


def kernel(x, W, b):
    B, d, n = x.shape
    d_out = W.shape[0]
    xt = jnp.transpose(x, (0, 2, 1))
    w1 = W[:, :d]
    w2 = W[:, d:]
    b2 = jnp.reshape(b, (1, d_out))

    grid = (B, n // ROWS)
    idx, h, g = pl.pallas_call(
        functools.partial(_topk_kernel, n=n, d=d, k=K),
        grid=grid,
        in_specs=[
            pl.BlockSpec((1, ROWS, d), lambda bi, ti: (bi, ti, 0)),
            pl.BlockSpec((1, n, d), lambda bi, ti: (bi, 0, 0)),
            pl.BlockSpec((d_out, d), lambda bi, ti: (0, 0)),
            pl.BlockSpec((d_out, d), lambda bi, ti: (0, 0)),
            pl.BlockSpec((1, d_out), lambda bi, ti: (0, 0)),
        ],
        out_specs=[
            pl.BlockSpec((1, ROWS, K), lambda bi, ti: (bi, ti, 0)),
            pl.BlockSpec((1, ROWS, d_out), lambda bi, ti: (bi, ti, 0)),
            pl.BlockSpec((1, ROWS, d_out), lambda bi, ti: (bi, ti, 0)),
        ],
        out_shape=[
            jax.ShapeDtypeStruct((B, n, K), jnp.int32),
            jax.ShapeDtypeStruct((B, n, d_out), jnp.float32),
            jax.ShapeDtypeStruct((B, n, d_out), jnp.float32),
        ],
    )(xt, xt, w1, w2, b2)

    total = B * n
    sc = _make_sc_gather_max(total, d_out, K)
    out = sc(jnp.reshape(idx, (total * K,)),
             jnp.reshape(h, (total, d_out)),
             jnp.reshape(g, (total, d_out)))
    out = jnp.reshape(out, (B, n, d_out))
    return jnp.transpose(out, (0, 2, 1))


# speedup vs baseline: 1.2682x; 1.0506x over previous
import functools

import jax
import jax.numpy as jnp
from jax import lax
from jax.experimental import pallas as pl
from jax.experimental.pallas import tpu as pltpu
from jax.experimental.pallas import tpu_sc as plsc

K = 16
ROWS = 512


def _topk_kernel(xr_ref, xf_ref, w1_ref, w2_ref, b_ref,
                 idx_ref, h_ref, g_ref, *, n, d, k):
    bi = pl.program_id(0)
    xr = xr_ref[0]
    xf = xf_ref[0]
    w1 = w1_ref[...]
    w2 = w2_ref[...]
    bias = b_ref[...]

    r = xr.shape[0]

    xx_r = jnp.sum(xr * xr, axis=1, keepdims=True)
    xx_f = jax.lax.dot_general(
        jnp.ones((1, d), jnp.float32), xf * xf, (((1,), (1,)), ((), ())),
        precision=jax.lax.Precision.HIGHEST,
        preferred_element_type=jnp.float32)
    a = jax.lax.dot_general(
        xr, xf, (((1,), (1,)), ((), ())),
        preferred_element_type=jnp.float32)
    scores = -xx_r - (-2.0 * a) - xx_f

    iota = jax.lax.broadcasted_iota(jnp.int32, (r, n), 1)
    kiota = jax.lax.broadcasted_iota(jnp.int32, (r, k), 1)

    def body(j, carry):
        scores, idxs = carry
        m = jnp.max(scores, axis=1, keepdims=True)
        eq = scores == m
        idx = jnp.min(jnp.where(eq, iota, n), axis=1, keepdims=True)
        scores = jnp.where(eq, -jnp.inf, scores)
        idxs = jnp.where(kiota == j, idx, idxs)
        return scores, idxs

    idxs0 = jnp.zeros((r, k), dtype=jnp.int32)
    _, idxs = jax.lax.fori_loop(0, k, body, (scores, idxs0))

    idx_ref[0] = idxs + bi * n
    h_ref[0] = jax.lax.dot_general(
        xr, w1, (((1,), (1,)), ((), ())),
        preferred_element_type=jnp.float32)
    g_ref[0] = jax.lax.dot_general(
        xr, w2 - w1, (((1,), (1,)), ((), ())),
        preferred_element_type=jnp.float32) + bias


def _make_sc_gather_max(total, d_out, k):
    info = plsc.get_sparse_core_info()
    nw = info.num_cores * info.num_subcores
    per_w = total // nw
    chunk = 8
    n_chunks = per_w // chunk
    mesh = plsc.VectorSubcoreMesh(core_axis_name="c", subcore_axis_name="s")

    @functools.partial(
        pl.kernel, mesh=mesh,
        out_type=jax.ShapeDtypeStruct((total, d_out), jnp.float32),
        scratch_types=[
            pltpu.VMEM((chunk * k,), jnp.int32),
            pltpu.VMEM((chunk * k, d_out), jnp.float32),
            pltpu.VMEM((chunk, d_out), jnp.float32),
            pltpu.VMEM((chunk, d_out), jnp.float32),
            pltpu.SemaphoreType.DMA,
        ],
    )
    def sc_gather_max(idx_hbm, h_hbm, g_hbm, out_hbm,
                      idx_v, rows_v, g_v, out_v, sem):
        wid = lax.axis_index("s") * info.num_cores + lax.axis_index("c")
        base = wid * per_w

        def chunk_body(c, _):
            cbase = base + c * chunk
            pltpu.sync_copy(idx_hbm.at[pl.ds(cbase * k, chunk * k)], idx_v)
            pltpu.async_copy(h_hbm.at[idx_v], rows_v, sem).wait()
            pltpu.sync_copy(g_hbm.at[pl.ds(cbase, chunk)], g_v)

            def point_body(p, _):
                for cg in range(d_out // 16):
                    acc = rows_v[p * k, pl.ds(cg * 16, 16)]
                    for j in range(1, k):
                        acc = jnp.maximum(
                            acc, rows_v[p * k + j, pl.ds(cg * 16, 16)])
                    out_v[p, pl.ds(cg * 16, 16)] = (
                        acc + g_v[p, pl.ds(cg * 16, 16)])
                return 0

            lax.fori_loop(0, chunk, point_body, 0)
            pltpu.sync_copy(out_v, out_hbm.at[pl.ds(cbase, chunk)])
            return 0

        lax.fori_loop(0, n_chunks, chunk_body, 0)

    return sc_gather_max


def kernel(x, W, b):
    B, d, n = x.shape
    d_out = W.shape[0]
    xt = jnp.transpose(x, (0, 2, 1))
    w1 = W[:, :d]
    w2 = W[:, d:]
    b2 = jnp.reshape(b, (1, d_out))

    grid = (B, n // ROWS)
    idx, h, g = pl.pallas_call(
        functools.partial(_topk_kernel, n=n, d=d, k=K),
        grid=grid,
        in_specs=[
            pl.BlockSpec((1, ROWS, d), lambda bi, ti: (bi, ti, 0)),
            pl.BlockSpec((1, n, d), lambda bi, ti: (bi, 0, 0)),
            pl.BlockSpec((d_out, d), lambda bi, ti: (0, 0)),
            pl.BlockSpec((d_out, d), lambda bi, ti: (0, 0)),
            pl.BlockSpec((1, d_out), lambda bi, ti: (0, 0)),
        ],
        out_specs=[
            pl.BlockSpec((1, ROWS, K), lambda bi, ti: (bi, ti, 0)),
            pl.BlockSpec((1, ROWS, d_out), lambda bi, ti: (bi, ti, 0)),
            pl.BlockSpec((1, ROWS, d_out), lambda bi, ti: (bi, ti, 0)),
        ],
        out_shape=[
            jax.ShapeDtypeStruct((B, n, K), jnp.int32),
            jax.ShapeDtypeStruct((B, n, d_out), jnp.float32),
            jax.ShapeDtypeStruct((B, n, d_out), jnp.float32),
        ],
    )(xt, xt, w1, w2, b2)

    total = B * n
    sc = _make_sc_gather_max(total, d_out, K)
    out = sc(jnp.reshape(idx, (total * K,)),
             jnp.reshape(h, (total, d_out)),
             jnp.reshape(g, (total, d_out)))
    out = jnp.reshape(out, (B, n, d_out))
    return jnp.transpose(out, (0, 2, 1))


# speedup vs baseline: 1.2855x; 1.0136x over previous
import functools

import jax
import jax.numpy as jnp
from jax import lax
from jax.experimental import pallas as pl
from jax.experimental.pallas import tpu as pltpu
from jax.experimental.pallas import tpu_sc as plsc

K = 16
ROWS = 1024


def _topk_kernel(xr_ref, xf_ref, w1_ref, w2_ref, b_ref,
                 idx_ref, h_ref, g_ref, *, n, d, k):
    bi = pl.program_id(0)
    xr = xr_ref[0]
    xf = xf_ref[0]
    w1 = w1_ref[...]
    w2 = w2_ref[...]
    bias = b_ref[...]

    r = xr.shape[0]

    xx_r = jnp.sum(xr * xr, axis=1, keepdims=True)
    xx_f = jax.lax.dot_general(
        jnp.ones((1, d), jnp.float32), xf * xf, (((1,), (1,)), ((), ())),
        precision=jax.lax.Precision.HIGHEST,
        preferred_element_type=jnp.float32)
    a = jax.lax.dot_general(
        xr, xf, (((1,), (1,)), ((), ())),
        preferred_element_type=jnp.float32)
    scores = -xx_r - (-2.0 * a) - xx_f

    iota = jax.lax.broadcasted_iota(jnp.int32, (r, n), 1)
    kiota = jax.lax.broadcasted_iota(jnp.int32, (r, k), 1)

    def body(j, carry):
        scores, idxs = carry
        m = jnp.max(scores, axis=1, keepdims=True)
        eq = scores == m
        idx = jnp.min(jnp.where(eq, iota, n), axis=1, keepdims=True)
        scores = jnp.where(eq, -jnp.inf, scores)
        idxs = jnp.where(kiota == j, idx, idxs)
        return scores, idxs

    idxs0 = jnp.zeros((r, k), dtype=jnp.int32)
    _, idxs = jax.lax.fori_loop(0, k, body, (scores, idxs0))

    idx_ref[0] = idxs + bi * n
    h_ref[0] = jax.lax.dot_general(
        xr, w1, (((1,), (1,)), ((), ())),
        preferred_element_type=jnp.float32)
    g_ref[0] = jax.lax.dot_general(
        xr, w2 - w1, (((1,), (1,)), ((), ())),
        preferred_element_type=jnp.float32) + bias


def _make_sc_gather_max(total, d_out, k):
    info = plsc.get_sparse_core_info()
    nw = info.num_cores * info.num_subcores
    per_w = total // nw
    chunk = 8
    n_chunks = per_w // chunk
    mesh = plsc.VectorSubcoreMesh(core_axis_name="c", subcore_axis_name="s")

    @functools.partial(
        pl.kernel, mesh=mesh,
        out_type=jax.ShapeDtypeStruct((total, d_out), jnp.float32),
        scratch_types=[
            pltpu.VMEM((chunk * k,), jnp.int32),
            pltpu.VMEM((chunk * k, d_out), jnp.float32),
            pltpu.VMEM((chunk, d_out), jnp.float32),
            pltpu.VMEM((chunk, d_out), jnp.float32),
            pltpu.SemaphoreType.DMA,
        ],
    )
    def sc_gather_max(idx_hbm, h_hbm, g_hbm, out_hbm,
                      idx_v, rows_v, g_v, out_v, sem):
        wid = lax.axis_index("s") * info.num_cores + lax.axis_index("c")
        base = wid * per_w

        def chunk_body(c, _):
            cbase = base + c * chunk
            pltpu.sync_copy(idx_hbm.at[pl.ds(cbase * k, chunk * k)], idx_v)
            pltpu.async_copy(h_hbm.at[idx_v], rows_v, sem).wait()
            pltpu.sync_copy(g_hbm.at[pl.ds(cbase, chunk)], g_v)

            def point_body(p, _):
                for cg in range(d_out // 16):
                    acc = rows_v[p * k, pl.ds(cg * 16, 16)]
                    for j in range(1, k):
                        acc = jnp.maximum(
                            acc, rows_v[p * k + j, pl.ds(cg * 16, 16)])
                    out_v[p, pl.ds(cg * 16, 16)] = (
                        acc + g_v[p, pl.ds(cg * 16, 16)])
                return 0

            lax.fori_loop(0, chunk, point_body, 0)
            pltpu.sync_copy(out_v, out_hbm.at[pl.ds(cbase, chunk)])
            return 0

        lax.fori_loop(0, n_chunks, chunk_body, 0)

    return sc_gather_max


def kernel(x, W, b):
    B, d, n = x.shape
    d_out = W.shape[0]
    xt = jnp.transpose(x, (0, 2, 1))
    w1 = W[:, :d]
    w2 = W[:, d:]
    b2 = jnp.reshape(b, (1, d_out))

    grid = (B, n // ROWS)
    idx, h, g = pl.pallas_call(
        functools.partial(_topk_kernel, n=n, d=d, k=K),
        grid=grid,
        in_specs=[
            pl.BlockSpec((1, ROWS, d), lambda bi, ti: (bi, ti, 0)),
            pl.BlockSpec((1, n, d), lambda bi, ti: (bi, 0, 0)),
            pl.BlockSpec((d_out, d), lambda bi, ti: (0, 0)),
            pl.BlockSpec((d_out, d), lambda bi, ti: (0, 0)),
            pl.BlockSpec((1, d_out), lambda bi, ti: (0, 0)),
        ],
        out_specs=[
            pl.BlockSpec((1, ROWS, K), lambda bi, ti: (bi, ti, 0)),
            pl.BlockSpec((1, ROWS, d_out), lambda bi, ti: (bi, ti, 0)),
            pl.BlockSpec((1, ROWS, d_out), lambda bi, ti: (bi, ti, 0)),
        ],
        out_shape=[
            jax.ShapeDtypeStruct((B, n, K), jnp.int32),
            jax.ShapeDtypeStruct((B, n, d_out), jnp.float32),
            jax.ShapeDtypeStruct((B, n, d_out), jnp.float32),
        ],
    )(xt, xt, w1, w2, b2)

    total = B * n
    sc = _make_sc_gather_max(total, d_out, K)
    out = sc(jnp.reshape(idx, (total * K,)),
             jnp.reshape(h, (total, d_out)),
             jnp.reshape(g, (total, d_out)))
    out = jnp.reshape(out, (B, n, d_out))
    return jnp.transpose(out, (0, 2, 1))


# speedup vs baseline: 1.2956x; 1.0079x over previous
import functools

import jax
import jax.numpy as jnp
from jax import lax
from jax.experimental import pallas as pl
from jax.experimental.pallas import tpu as pltpu
from jax.experimental.pallas import tpu_sc as plsc

K = 16
ROWS = 1024


def _topk_kernel(xr_ref, xf_ref, w1_ref, w2_ref, b_ref,
                 idx_ref, h_ref, g_ref, *, n, d, k):
    bi = pl.program_id(0)
    xr = xr_ref[0]
    xf = xf_ref[0]
    w1 = w1_ref[...]
    w2 = w2_ref[...]
    bias = b_ref[...]

    r = xr.shape[0]

    xx_r = jnp.sum(xr * xr, axis=1, keepdims=True)
    xx_f = jax.lax.dot_general(
        jnp.ones((1, d), jnp.float32), xf * xf, (((1,), (1,)), ((), ())),
        precision=jax.lax.Precision.HIGHEST,
        preferred_element_type=jnp.float32)
    a = jax.lax.dot_general(
        xr, xf, (((1,), (1,)), ((), ())),
        preferred_element_type=jnp.float32)
    scores = -xx_r - (-2.0 * a) - xx_f

    iota = jax.lax.broadcasted_iota(jnp.int32, (r, n), 1)
    kiota = jax.lax.broadcasted_iota(jnp.int32, (r, k), 1)

    def body(j, carry):
        m_prev, i_prev, idxs = carry
        alive = (scores < m_prev) | ((scores == m_prev) & (iota > i_prev))
        cand = jnp.where(alive, scores, -jnp.inf)
        m = jnp.max(cand, axis=1, keepdims=True)
        idx = jnp.min(jnp.where(cand == m, iota, n), axis=1, keepdims=True)
        idxs = jnp.where(kiota == j, idx, idxs)
        return m, idx, idxs

    idxs0 = jnp.zeros((r, k), dtype=jnp.int32)
    m0 = jnp.full((r, 1), jnp.inf, dtype=jnp.float32)
    i0 = jnp.full((r, 1), -1, dtype=jnp.int32)
    _, _, idxs = jax.lax.fori_loop(0, k, body, (m0, i0, idxs0))

    idx_ref[0] = idxs + bi * n
    h_ref[0] = jax.lax.dot_general(
        xr, w1, (((1,), (1,)), ((), ())),
        preferred_element_type=jnp.float32)
    g_ref[0] = jax.lax.dot_general(
        xr, w2 - w1, (((1,), (1,)), ((), ())),
        preferred_element_type=jnp.float32) + bias


def _make_sc_gather_max(total, d_out, k):
    info = plsc.get_sparse_core_info()
    nw = info.num_cores * info.num_subcores
    per_w = total // nw
    chunk = 8
    n_chunks = per_w // chunk
    mesh = plsc.VectorSubcoreMesh(core_axis_name="c", subcore_axis_name="s")

    @functools.partial(
        pl.kernel, mesh=mesh,
        out_type=jax.ShapeDtypeStruct((total, d_out), jnp.float32),
        scratch_types=[
            pltpu.VMEM((chunk * k,), jnp.int32),
            pltpu.VMEM((chunk * k, d_out), jnp.float32),
            pltpu.VMEM((chunk, d_out), jnp.float32),
            pltpu.VMEM((chunk, d_out), jnp.float32),
            pltpu.SemaphoreType.DMA,
        ],
    )
    def sc_gather_max(idx_hbm, h_hbm, g_hbm, out_hbm,
                      idx_v, rows_v, g_v, out_v, sem):
        wid = lax.axis_index("s") * info.num_cores + lax.axis_index("c")
        base = wid * per_w

        def chunk_body(c, _):
            cbase = base + c * chunk
            pltpu.sync_copy(idx_hbm.at[pl.ds(cbase * k, chunk * k)], idx_v)
            pltpu.async_copy(h_hbm.at[idx_v], rows_v, sem).wait()
            pltpu.sync_copy(g_hbm.at[pl.ds(cbase, chunk)], g_v)

            def point_body(p, _):
                for cg in range(d_out // 16):
                    acc = rows_v[p * k, pl.ds(cg * 16, 16)]
                    for j in range(1, k):
                        acc = jnp.maximum(
                            acc, rows_v[p * k + j, pl.ds(cg * 16, 16)])
                    out_v[p, pl.ds(cg * 16, 16)] = (
                        acc + g_v[p, pl.ds(cg * 16, 16)])
                return 0

            lax.fori_loop(0, chunk, point_body, 0)
            pltpu.sync_copy(out_v, out_hbm.at[pl.ds(cbase, chunk)])
            return 0

        lax.fori_loop(0, n_chunks, chunk_body, 0)

    return sc_gather_max


def kernel(x, W, b):
    B, d, n = x.shape
    d_out = W.shape[0]
    xt = jnp.transpose(x, (0, 2, 1))
    w1 = W[:, :d]
    w2 = W[:, d:]
    b2 = jnp.reshape(b, (1, d_out))

    grid = (B, n // ROWS)
    idx, h, g = pl.pallas_call(
        functools.partial(_topk_kernel, n=n, d=d, k=K),
        grid=grid,
        in_specs=[
            pl.BlockSpec((1, ROWS, d), lambda bi, ti: (bi, ti, 0)),
            pl.BlockSpec((1, n, d), lambda bi, ti: (bi, 0, 0)),
            pl.BlockSpec((d_out, d), lambda bi, ti: (0, 0)),
            pl.BlockSpec((d_out, d), lambda bi, ti: (0, 0)),
            pl.BlockSpec((1, d_out), lambda bi, ti: (0, 0)),
        ],
        out_specs=[
            pl.BlockSpec((1, ROWS, K), lambda bi, ti: (bi, ti, 0)),
            pl.BlockSpec((1, ROWS, d_out), lambda bi, ti: (bi, ti, 0)),
            pl.BlockSpec((1, ROWS, d_out), lambda bi, ti: (bi, ti, 0)),
        ],
        out_shape=[
            jax.ShapeDtypeStruct((B, n, K), jnp.int32),
            jax.ShapeDtypeStruct((B, n, d_out), jnp.float32),
            jax.ShapeDtypeStruct((B, n, d_out), jnp.float32),
        ],
    )(xt, xt, w1, w2, b2)

    total = B * n
    sc = _make_sc_gather_max(total, d_out, K)
    out = sc(jnp.reshape(idx, (total * K,)),
             jnp.reshape(h, (total, d_out)),
             jnp.reshape(g, (total, d_out)))
    out = jnp.reshape(out, (B, n, d_out))
    return jnp.transpose(out, (0, 2, 1))
